# baseline (speedup 1.0000x reference)
import jax
import jax.numpy as jnp
from jax.experimental import pallas as pl

_B = 512
_C = 512
_HW = 196
_G = 1024
_TB = 64
_EPS = 1e-8
_AVG = 0.5


def _body(m0_ref, m1_ref, w1_ref, w2_ref, xsun_ref, xgl_ref, xson_ref):
    s = (jnp.sum(m0_ref[...], axis=0) + jnp.sum(m1_ref[...], axis=0)) * (1.0 / _HW)
    xsun_ref[...] = s
    xlog = jnp.log(jnp.tanh(jnp.maximum(s, 0.0) + _EPS))
    gl = jax.lax.dot_general(
        xlog, w1_ref[...], (((1,), (1,)), ((), ())),
        preferred_element_type=jnp.float32)
    xgl_ref[...] = gl
    vote = (jnp.exp(gl) - _EPS) * w2_ref[...]
    dense = jnp.sum(vote, axis=1, keepdims=True)
    absv = jnp.abs(vote)
    iota = jax.lax.broadcasted_iota(jnp.int32, vote.shape, 1)
    acc = jnp.zeros((vote.shape[0], 1), jnp.float32)
    cols = []
    for _ in range(8):
        mx = jnp.max(absv, axis=1, keepdims=True)
        idx = jnp.min(jnp.where(absv == mx, iota, _G), axis=1, keepdims=True)
        hit = iota == idx
        acc = acc + jnp.sum(jnp.where(hit, vote, 0.0), axis=1, keepdims=True)
        cols.append(acc + _AVG)
        absv = jnp.where(hit, -1.0, absv)
    cols.append(dense + _AVG)
    xson_ref[...] = jnp.concatenate(cols, axis=1)


def kernel(maps, W1, W2):
    maps_t = maps.transpose(2, 3, 0, 1).reshape(_HW, _B, _C)
    xsun, xgl, xson = pl.pallas_call(
        _body,
        grid=(_B // _TB,),
        in_specs=[
            pl.BlockSpec((_HW // 2, _TB, _C), lambda i: (0, i, 0)),
            pl.BlockSpec((_HW // 2, _TB, _C), lambda i: (1, i, 0)),
            pl.BlockSpec((_G, _C), lambda i: (0, 0)),
            pl.BlockSpec((1, _G), lambda i: (0, 0)),
        ],
        out_specs=[
            pl.BlockSpec((_TB, _C), lambda i: (i, 0)),
            pl.BlockSpec((_TB, _G), lambda i: (i, 0)),
            pl.BlockSpec((_TB, 9), lambda i: (i, 0)),
        ],
        out_shape=[
            jax.ShapeDtypeStruct((_B, _C), jnp.float32),
            jax.ShapeDtypeStruct((_B, _G), jnp.float32),
            jax.ShapeDtypeStruct((_B, 9), jnp.float32),
        ],
    )(maps_t, maps_t, W1, W2)
    return (xsun, xgl, xson)
